# baseline (speedup 1.0000x reference)
import jax
import jax.numpy as jnp
from jax.experimental import pallas as pl
from jax.experimental.pallas import tpu as pltpu

_B, _D, _E = 8192, 4096, 64
_H1, _H2, _H3 = 128, 256, 128
_R = 1024
_N = _B // _R
_DH = _D // 4

_DN = (((1,), (1,)), ((), ()))


def _gating_kernel(xa_ref, xb_ref, xc_ref, xd_ref, w1_ref, b1_ref, w2_ref, b2_ref, w3_ref,
                   b3_ref, w4_ref, b4_ref, out_ref, acc_ref):
    step = pl.program_id(0)

    h = (jax.lax.dot_general(xa_ref[...], w1_ref[:, :_DH], _DN,
                             preferred_element_type=jnp.float32)
         + jax.lax.dot_general(xb_ref[...], w1_ref[:, _DH:2 * _DH], _DN,
                               preferred_element_type=jnp.float32)
         + jax.lax.dot_general(xc_ref[...], w1_ref[:, 2 * _DH:3 * _DH], _DN,
                               preferred_element_type=jnp.float32)
         + jax.lax.dot_general(xd_ref[...], w1_ref[:, 3 * _DH:], _DN,
                               preferred_element_type=jnp.float32)
         + b1_ref[...])
    h = jnp.maximum(h, 0.0)
    h = jax.lax.dot_general(h, w2_ref[...], _DN,
                            preferred_element_type=jnp.float32) + b2_ref[...]
    h = jnp.where(h >= 0, h, 0.01 * h)
    h = jax.lax.dot_general(h, w3_ref[...], _DN,
                            preferred_element_type=jnp.float32) + b3_ref[...]
    h = jnp.where(h >= 0, h, 0.01 * h)
    logits = jax.lax.dot_general(h, w4_ref[...], _DN,
                                 preferred_element_type=jnp.float32) + b4_ref[...]

    col = jax.lax.broadcasted_iota(jnp.int32, (_R, _E), 1)
    m1 = jnp.max(logits, axis=1, keepdims=True)
    i1 = jnp.min(jnp.where(logits == m1, col, _E), axis=1, keepdims=True)
    masked = jnp.where(col == i1, -jnp.inf, logits)
    m2 = jnp.max(masked, axis=1, keepdims=True)
    i2 = jnp.min(jnp.where(masked == m2, col, _E), axis=1, keepdims=True)

    psum = jnp.sum(m1) + jnp.sum(m2)
    prev = jnp.where(step == 0, 0.0, acc_ref[0])
    total = prev + psum
    acc_ref[0] = total

    out_ref[...] = jnp.zeros((_R, _E), jnp.float32)

    @pl.when(step == _N - 1)
    def _write_row0():
        lane = jax.lax.broadcasted_iota(jnp.int32, (1, _E), 1)
        row = (jnp.where(lane == i1[0:1], m1[0:1] / total, 0.0)
               + jnp.where(lane == i2[0:1], m2[0:1] / total, 0.0))
        out_ref[0:1, :] = row


def kernel(x, W1, b1, W2, b2, W3, b3, W4, b4):
    b1r = b1.reshape(1, _H1)
    b2r = b2.reshape(1, _H2)
    b3r = b3.reshape(1, _H3)
    b4r = b4.reshape(1, _E)
    revl = lambda i: (_N - 1 - i, 0)
    revb = lambda i: (_N - 1 - i, 1)
    revc = lambda i: (_N - 1 - i, 2)
    revd = lambda i: (_N - 1 - i, 3)
    fixed = lambda i: (0, 0)
    return pl.pallas_call(
        _gating_kernel,
        grid=(_N,),
        in_specs=[
            pl.BlockSpec((_R, _DH), revl),
            pl.BlockSpec((_R, _DH), revb),
            pl.BlockSpec((_R, _DH), revc),
            pl.BlockSpec((_R, _DH), revd),
            pl.BlockSpec((_H1, _D), fixed),
            pl.BlockSpec((1, _H1), fixed),
            pl.BlockSpec((_H2, _H1), fixed),
            pl.BlockSpec((1, _H2), fixed),
            pl.BlockSpec((_H3, _H2), fixed),
            pl.BlockSpec((1, _H3), fixed),
            pl.BlockSpec((_E, _H3), fixed),
            pl.BlockSpec((1, _E), fixed),
        ],
        out_specs=pl.BlockSpec((_R, _E), revl),
        out_shape=jax.ShapeDtypeStruct((_B, _E), jnp.float32),
        scratch_shapes=[pltpu.SMEM((1,), jnp.float32)],
    )(x, x, x, x, W1, b1r, W2, b2r, W3, b3r, W4, b4r)


# speedup vs baseline: 1.0183x; 1.0183x over previous
import jax
import jax.numpy as jnp
from jax.experimental import pallas as pl
from jax.experimental.pallas import tpu as pltpu

_B, _D, _E = 8192, 4096, 64
_H1, _H2, _H3 = 128, 256, 128
_R = 1024
_N = _B // _R
_DH = _D // 2

_DN = (((1,), (1,)), ((), ()))


def _gating_kernel(xl_ref, xr_ref, w1_ref, b1_ref, w2_ref, b2_ref, w3_ref,
                   b3_ref, w4_ref, b4_ref, out_ref, acc_ref):
    step = pl.program_id(0)

    h = (jax.lax.dot_general(xl_ref[...], w1_ref[:, :_DH], _DN,
                             preferred_element_type=jnp.float32)
         + jax.lax.dot_general(xr_ref[...], w1_ref[:, _DH:], _DN,
                               preferred_element_type=jnp.float32)
         + b1_ref[...])
    h = jnp.maximum(h, 0.0)
    h = jax.lax.dot_general(h, w2_ref[...], _DN,
                            preferred_element_type=jnp.float32) + b2_ref[...]
    h = jnp.where(h >= 0, h, 0.01 * h)
    h = jax.lax.dot_general(h, w3_ref[...], _DN,
                            preferred_element_type=jnp.float32) + b3_ref[...]
    h = jnp.where(h >= 0, h, 0.01 * h)
    logits = jax.lax.dot_general(h, w4_ref[...], _DN,
                                 preferred_element_type=jnp.float32) + b4_ref[...]

    col = jax.lax.broadcasted_iota(jnp.int32, (_R, _E), 1)
    m1 = jnp.max(logits, axis=1, keepdims=True)
    i1 = jnp.min(jnp.where(logits == m1, col, _E), axis=1, keepdims=True)
    masked = jnp.where(col == i1, -jnp.inf, logits)
    m2 = jnp.max(masked, axis=1, keepdims=True)
    i2 = jnp.min(jnp.where(masked == m2, col, _E), axis=1, keepdims=True)

    psum = jnp.sum(m1) + jnp.sum(m2)
    prev = jnp.where(step == 0, 0.0, acc_ref[0])
    total = prev + psum
    acc_ref[0] = total

    out_ref[...] = jnp.zeros((_R, _E), jnp.float32)

    @pl.when(step == _N - 1)
    def _write_row0():
        lane = jax.lax.broadcasted_iota(jnp.int32, (1, _E), 1)
        row = (jnp.where(lane == i1[0:1], m1[0:1] / total, 0.0)
               + jnp.where(lane == i2[0:1], m2[0:1] / total, 0.0))
        out_ref[0:1, :] = row


def kernel(x, W1, b1, W2, b2, W3, b3, W4, b4):
    b1r = b1.reshape(1, _H1)
    b2r = b2.reshape(1, _H2)
    b3r = b3.reshape(1, _H3)
    b4r = b4.reshape(1, _E)
    revl = lambda i: (_N - 1 - i, 0)
    revr = lambda i: (_N - 1 - i, 1)
    fixed = lambda i: (0, 0)
    return pl.pallas_call(
        _gating_kernel,
        grid=(_N,),
        in_specs=[
            pl.BlockSpec((_R, _DH), revl),
            pl.BlockSpec((_R, _DH), revr),
            pl.BlockSpec((_H1, _D), fixed),
            pl.BlockSpec((1, _H1), fixed),
            pl.BlockSpec((_H2, _H1), fixed),
            pl.BlockSpec((1, _H2), fixed),
            pl.BlockSpec((_H3, _H2), fixed),
            pl.BlockSpec((1, _H3), fixed),
            pl.BlockSpec((_E, _H3), fixed),
            pl.BlockSpec((1, _E), fixed),
        ],
        out_specs=pl.BlockSpec((_R, _E), revl),
        out_shape=jax.ShapeDtypeStruct((_B, _E), jnp.float32),
        scratch_shapes=[pltpu.SMEM((1,), jnp.float32)],
    )(x, x, W1, b1r, W2, b2r, W3, b3r, W4, b4r)
